# baseline (speedup 1.0000x reference)
import functools

import jax
import jax.numpy as jnp
from jax import lax
from jax.experimental import pallas as pl
from jax.experimental.pallas import tpu as pltpu
from jax.experimental.pallas import tpu_sc as plsc

INPUT_DIM = 768
META_DIM = 16
HID = 32
NUM_ACTIONS = 8
NUM_INPUTS = 2 * INPUT_DIM + META_DIM + 1
N_AC = INPUT_DIM + META_DIM
NP = 1664
ZW = 32
B = 16384
SUB_B = 512
N_STREAMS = 4
STEP_B = SUB_B * N_STREAMS

NW = 32
ROWS_W = B // NW
LN2 = 0.6931471805599453
SQRT2 = 1.4142135623730951


def _assemble_body(aw1, ab1, aw2, ab2, aw3, ab3, bw1, bb1, bw2, bb2, bw3, bb3,
                   acw1, acb1, acw2, acb2, acw3, acb3, bcw1, bcb1, bcw2, bcb2,
                   bcw3, bcb3, w1o, b1o, w2o, b2o, w3o, b3o):
    f32 = jnp.float32
    bf16 = jnp.bfloat16
    w1o[...] = jnp.zeros((NUM_INPUTS, 4 * HID), bf16)
    w1o[0:N_AC, 0:HID] = aw1[...].astype(bf16)
    w1o[0:INPUT_DIM, HID:2 * HID] = bw1[0:INPUT_DIM, :].astype(bf16)
    w1o[N_AC:N_AC + INPUT_DIM, HID:2 * HID] = (
        bw1[INPUT_DIM:2 * INPUT_DIM, :].astype(bf16))
    w1o[0:NUM_INPUTS, 2 * HID:3 * HID] = acw1[...].astype(bf16)
    w1o[0:NUM_INPUTS, 3 * HID:4 * HID] = bcw1[...].astype(bf16)
    w2o[...] = jnp.zeros((4 * HID, 4 * HID), f32)
    w2o[0:HID, 0:HID] = aw2[...]
    w2o[HID:2 * HID, HID:2 * HID] = bw2[...]
    w2o[2 * HID:3 * HID, 2 * HID:3 * HID] = acw2[...]
    w2o[3 * HID:4 * HID, 3 * HID:4 * HID] = bcw2[...]
    w3o[...] = jnp.zeros((4 * HID, ZW), f32)
    w3o[0:HID, 0:NUM_ACTIONS] = aw3[...]
    w3o[HID:2 * HID, NUM_ACTIONS:2 * NUM_ACTIONS] = bw3[...]
    w3o[2 * HID:3 * HID, 16:17] = acw3[...]
    w3o[3 * HID:4 * HID, 17:18] = bcw3[...]
    b1o[0:1, 0:HID] = ab1[...].reshape(1, HID)
    b1o[0:1, HID:2 * HID] = bb1[...].reshape(1, HID)
    b1o[0:1, 2 * HID:3 * HID] = acb1[...].reshape(1, HID)
    b1o[0:1, 3 * HID:4 * HID] = bcb1[...].reshape(1, HID)
    b2o[0:1, 0:HID] = ab2[...].reshape(1, HID)
    b2o[0:1, HID:2 * HID] = bb2[...].reshape(1, HID)
    b2o[0:1, 2 * HID:3 * HID] = acb2[...].reshape(1, HID)
    b2o[0:1, 3 * HID:4 * HID] = bcb2[...].reshape(1, HID)
    b3o[...] = jnp.zeros((1, ZW), f32)
    b3o[0:1, 0:NUM_ACTIONS] = ab3[...].reshape(1, NUM_ACTIONS)
    b3o[0:1, NUM_ACTIONS:2 * NUM_ACTIONS] = bb3[...].reshape(1, NUM_ACTIONS)
    b3o[0:1, 16:17] = acb3[...].reshape(1, 1)
    b3o[0:1, 17:18] = bcb3[...].reshape(1, 1)


def _dense_body(x0, x1, x2, x3, w1, b1, w2, b2, w3, b3, z_ref):
    for k, x_ref in enumerate((x0, x1, x2, x3)):
        x = x_ref[...]
        acc = jnp.dot(x, w1[...], preferred_element_type=jnp.float32)
        h1 = jnp.tanh(acc + b1[...])
        h2 = jnp.tanh(
            jnp.dot(h1, w2[...], preferred_element_type=jnp.float32) + b2[...])
        z = jnp.dot(h2, w3[...], preferred_element_type=jnp.float32) + b3[...]
        rows = pl.ds(k * SUB_B, SUB_B)
        z_ref[rows, 0:ZW] = z
        z_ref[rows, 18:19] = x[:, NUM_INPUTS - 1:NUM_INPUTS].astype(jnp.float32)


def _ln(s):
    i32 = jnp.int32
    f32 = jnp.float32
    bits = lax.bitcast_convert_type(s, i32)
    k = (bits >> 23) - 127
    man = lax.bitcast_convert_type(
        (bits & jnp.int32(0x007FFFFF)) | jnp.int32(0x3F800000), f32)
    adj = man > SQRT2
    man = jnp.where(adj, man * 0.5, man)
    kf = k.astype(f32) + jnp.where(adj, 1.0, 0.0).astype(f32)
    t = (man - 1.0) / (man + 1.0)
    t2 = t * t
    series = t * (2.0 + t2 * (2.0 / 3.0 + t2 * (2.0 / 5.0 + t2 * (2.0 / 7.0))))
    return kf * LN2 + series


def _combine_body(z_hbm, a_hbm, out_hbm, z_v, a_v, out_v):
    wid = lax.axis_index("s") * 2 + lax.axis_index("c")
    base = wid * ROWS_W
    pltpu.sync_copy(z_hbm.at[pl.ds(base, ROWS_W)], z_v)
    pltpu.sync_copy(a_hbm.at[pl.ds(base, ROWS_W)], a_v)

    lane = lax.iota(jnp.int32, 16)

    def group(g, _):
        rows = g * 16 + lane
        mind = plsc.load_gather(z_v, [rows, jnp.full((16,), 18, jnp.int32)])
        is_bob = (mind > 1.5).astype(jnp.int32)
        col0 = is_bob * NUM_ACTIONS
        l0 = plsc.load_gather(z_v, [rows, col0])
        m = l0
        for j in range(1, NUM_ACTIONS):
            lj = plsc.load_gather(z_v, [rows, col0 + j])
            m = jnp.maximum(m, lj)
        s = jnp.zeros((16,), jnp.float32)
        for j in range(NUM_ACTIONS):
            lj = plsc.load_gather(z_v, [rows, col0 + j])
            s = s + jnp.exp(lj - m)
        lse = m + _ln(s)
        a_vec = a_v[pl.ds(g * 16, 16)]
        sel = plsc.load_gather(z_v, [rows, col0 + a_vec])
        logp = sel - lse
        v = plsc.load_gather(z_v, [rows, 16 + is_bob])
        plsc.store_scatter(out_v, [rows * 2], logp)
        plsc.store_scatter(out_v, [rows * 2 + 1], v)
        return _

    lax.fori_loop(0, ROWS_W // 16, group, None)
    pltpu.sync_copy(out_v, out_hbm.at[pl.ds(base * 2, ROWS_W * 2)])


def _make_combine():
    return functools.partial(
        pl.kernel,
        out_type=jax.ShapeDtypeStruct((B * 2,), jnp.float32),
        mesh=plsc.VectorSubcoreMesh(core_axis_name="c", subcore_axis_name="s"),
        scratch_types=[
            pltpu.VMEM((ROWS_W, 128), jnp.float32),
            pltpu.VMEM((ROWS_W,), jnp.int32),
            pltpu.VMEM((ROWS_W * 2,), jnp.float32),
        ],
        compiler_params=pltpu.CompilerParams(needs_layout_passes=False),
    )(_combine_body)


def kernel(x, a, aw1, ab1, aw2, ab2, aw3, ab3, bw1, bb1, bw2, bb2, bw3, bb3,
           acw1, acb1, acw2, acb2, acw3, acb3, bcw1, bcb1, bcw2, bcb2, bcw3,
           bcb3):
    f32 = jnp.float32
    full = lambda s: pl.BlockSpec(s, lambda: (0,) * len(s))
    w1, b1, w2, b2, w3, b3 = pl.pallas_call(
        _assemble_body,
        in_specs=[full(t.shape) for t in (
            aw1, ab1, aw2, ab2, aw3, ab3, bw1, bb1, bw2, bb2, bw3, bb3,
            acw1, acb1, acw2, acb2, acw3, acb3, bcw1, bcb1, bcw2, bcb2,
            bcw3, bcb3)],
        out_specs=[full((NUM_INPUTS, 4 * HID)), full((1, 4 * HID)),
                   full((4 * HID, 4 * HID)), full((1, 4 * HID)),
                   full((4 * HID, ZW)), full((1, ZW))],
        out_shape=[jax.ShapeDtypeStruct((NUM_INPUTS, 4 * HID), jnp.bfloat16),
                   jax.ShapeDtypeStruct((1, 4 * HID), f32),
                   jax.ShapeDtypeStruct((4 * HID, 4 * HID), f32),
                   jax.ShapeDtypeStruct((1, 4 * HID), f32),
                   jax.ShapeDtypeStruct((4 * HID, ZW), f32),
                   jax.ShapeDtypeStruct((1, ZW), f32)],
    )(aw1, ab1, aw2, ab2, aw3, ab3, bw1, bb1, bw2, bb2, bw3, bb3,
      acw1, acb1, acw2, acb2, acw3, acb3, bcw1, bcb1, bcw2, bcb2, bcw3, bcb3)

    xp = x.astype(jnp.bfloat16)
    a1 = a.astype(jnp.int32)

    grid = (B // STEP_B,)
    xs = lambda k: pl.BlockSpec((SUB_B, NUM_INPUTS),
                                lambda i, kk=k: (N_STREAMS * i + kk, 0))
    z = pl.pallas_call(
        _dense_body,
        grid=grid,
        in_specs=[
            xs(0), xs(1), xs(2), xs(3),
            pl.BlockSpec((NUM_INPUTS, 4 * HID), lambda i: (0, 0)),
            pl.BlockSpec((1, 4 * HID), lambda i: (0, 0)),
            pl.BlockSpec((4 * HID, 4 * HID), lambda i: (0, 0)),
            pl.BlockSpec((1, 4 * HID), lambda i: (0, 0)),
            pl.BlockSpec((4 * HID, ZW), lambda i: (0, 0)),
            pl.BlockSpec((1, ZW), lambda i: (0, 0)),
        ],
        out_specs=pl.BlockSpec((STEP_B, 128), lambda i: (i, 0)),
        out_shape=jax.ShapeDtypeStruct((B, 128), f32),
    )(xp, xp, xp, xp, w1, b1, w2, b2, w3, b3)

    return _make_combine()(z, a1).reshape(B, 2)


# speedup vs baseline: 1.0667x; 1.0667x over previous
import functools

import jax
import jax.numpy as jnp
from jax import lax
from jax.experimental import pallas as pl
from jax.experimental.pallas import tpu as pltpu
from jax.experimental.pallas import tpu_sc as plsc

INPUT_DIM = 768
META_DIM = 16
HID = 32
NUM_ACTIONS = 8
NUM_INPUTS = 2 * INPUT_DIM + META_DIM + 1
N_AC = INPUT_DIM + META_DIM
MAIN_W = 1536
TAIL_W = NUM_INPUTS - MAIN_W
ZW = 32
B = 16384
SUB_B = 512
N_STREAMS = 4
STEP_B = SUB_B * N_STREAMS

NW = 32
ROWS_W = B // NW
LN2 = 0.6931471805599453
SQRT2 = 1.4142135623730951


def _assemble_body(aw1, ab1, aw2, ab2, aw3, ab3, bw1, bb1, bw2, bb2, bw3, bb3,
                   acw1, acb1, acw2, acb2, acw3, acb3, bcw1, bcb1, bcw2, bcb2,
                   bcw3, bcb3, w1o, wto, b1o, w2o, b2o, w3o, b3o):
    f32 = jnp.float32
    bf16 = jnp.bfloat16
    w1o[...] = jnp.zeros((MAIN_W, 4 * HID), bf16)
    w1o[0:N_AC, 0:HID] = aw1[...].astype(bf16)
    w1o[0:INPUT_DIM, HID:2 * HID] = bw1[0:INPUT_DIM, :].astype(bf16)
    w1o[N_AC:MAIN_W, HID:2 * HID] = (
        bw1[INPUT_DIM:INPUT_DIM + MAIN_W - N_AC, :].astype(bf16))
    w1o[0:MAIN_W, 2 * HID:3 * HID] = acw1[0:MAIN_W, :].astype(bf16)
    w1o[0:MAIN_W, 3 * HID:4 * HID] = bcw1[0:MAIN_W, :].astype(bf16)
    wto[...] = jnp.zeros((24, 4 * HID), f32)
    wto[0:TAIL_W - 1, HID:2 * HID] = (
        bw1[INPUT_DIM + MAIN_W - N_AC:2 * INPUT_DIM, :].astype(f32))
    wto[0:TAIL_W, 2 * HID:3 * HID] = acw1[MAIN_W:NUM_INPUTS, :]
    wto[0:TAIL_W, 3 * HID:4 * HID] = bcw1[MAIN_W:NUM_INPUTS, :]
    w2o[...] = jnp.zeros((4 * HID, 4 * HID), f32)
    w2o[0:HID, 0:HID] = aw2[...]
    w2o[HID:2 * HID, HID:2 * HID] = bw2[...]
    w2o[2 * HID:3 * HID, 2 * HID:3 * HID] = acw2[...]
    w2o[3 * HID:4 * HID, 3 * HID:4 * HID] = bcw2[...]
    w3o[...] = jnp.zeros((4 * HID, ZW), f32)
    w3o[0:HID, 0:NUM_ACTIONS] = aw3[...]
    w3o[HID:2 * HID, NUM_ACTIONS:2 * NUM_ACTIONS] = bw3[...]
    w3o[2 * HID:3 * HID, 16:17] = acw3[...]
    w3o[3 * HID:4 * HID, 17:18] = bcw3[...]
    b1o[0:1, 0:HID] = ab1[...].reshape(1, HID)
    b1o[0:1, HID:2 * HID] = bb1[...].reshape(1, HID)
    b1o[0:1, 2 * HID:3 * HID] = acb1[...].reshape(1, HID)
    b1o[0:1, 3 * HID:4 * HID] = bcb1[...].reshape(1, HID)
    b2o[0:1, 0:HID] = ab2[...].reshape(1, HID)
    b2o[0:1, HID:2 * HID] = bb2[...].reshape(1, HID)
    b2o[0:1, 2 * HID:3 * HID] = acb2[...].reshape(1, HID)
    b2o[0:1, 3 * HID:4 * HID] = bcb2[...].reshape(1, HID)
    b3o[...] = jnp.zeros((1, ZW), f32)
    b3o[0:1, 0:NUM_ACTIONS] = ab3[...].reshape(1, NUM_ACTIONS)
    b3o[0:1, NUM_ACTIONS:2 * NUM_ACTIONS] = bb3[...].reshape(1, NUM_ACTIONS)
    b3o[0:1, 16:17] = acb3[...].reshape(1, 1)
    b3o[0:1, 17:18] = bcb3[...].reshape(1, 1)


def _dense_body(x0, x1, x2, x3, xt, w1, wt, b1, w2, b2, w3, b3, z_ref):
    for k, x_ref in enumerate((x0, x1, x2, x3)):
        x = x_ref[...]
        tail = xt[pl.ds(k * SUB_B, SUB_B), :]
        acc = jnp.dot(x, w1[...], preferred_element_type=jnp.float32)
        acc += jnp.dot(tail, wt[0:TAIL_W, :],
                       preferred_element_type=jnp.float32)
        h1 = jnp.tanh(acc + b1[...])
        h2 = jnp.tanh(
            jnp.dot(h1, w2[...], preferred_element_type=jnp.float32) + b2[...])
        z = jnp.dot(h2, w3[...], preferred_element_type=jnp.float32) + b3[...]
        rows = pl.ds(k * SUB_B, SUB_B)
        z_ref[rows, 0:ZW] = z
        z_ref[rows, 18:19] = tail[:, TAIL_W - 1:TAIL_W]


def _ln(s):
    i32 = jnp.int32
    f32 = jnp.float32
    bits = lax.bitcast_convert_type(s, i32)
    k = (bits >> 23) - 127
    man = lax.bitcast_convert_type(
        (bits & jnp.int32(0x007FFFFF)) | jnp.int32(0x3F800000), f32)
    adj = man > SQRT2
    man = jnp.where(adj, man * 0.5, man)
    kf = k.astype(f32) + jnp.where(adj, 1.0, 0.0).astype(f32)
    t = (man - 1.0) / (man + 1.0)
    t2 = t * t
    series = t * (2.0 + t2 * (2.0 / 3.0 + t2 * (2.0 / 5.0 + t2 * (2.0 / 7.0))))
    return kf * LN2 + series


def _combine_body(z_hbm, a_hbm, out_hbm, z_v, a_v, out_v):
    wid = lax.axis_index("s") * 2 + lax.axis_index("c")
    base = wid * ROWS_W
    pltpu.sync_copy(z_hbm.at[pl.ds(base, ROWS_W)], z_v)
    pltpu.sync_copy(a_hbm.at[pl.ds(base, ROWS_W)], a_v)

    lane = lax.iota(jnp.int32, 16)

    def group(g, _):
        rows = g * 16 + lane
        mind = plsc.load_gather(z_v, [rows, jnp.full((16,), 18, jnp.int32)])
        is_bob = (mind > 1.5).astype(jnp.int32)
        col0 = is_bob * NUM_ACTIONS
        l0 = plsc.load_gather(z_v, [rows, col0])
        m = l0
        for j in range(1, NUM_ACTIONS):
            lj = plsc.load_gather(z_v, [rows, col0 + j])
            m = jnp.maximum(m, lj)
        s = jnp.zeros((16,), jnp.float32)
        for j in range(NUM_ACTIONS):
            lj = plsc.load_gather(z_v, [rows, col0 + j])
            s = s + jnp.exp(lj - m)
        lse = m + _ln(s)
        a_vec = a_v[pl.ds(g * 16, 16)]
        sel = plsc.load_gather(z_v, [rows, col0 + a_vec])
        logp = sel - lse
        v = plsc.load_gather(z_v, [rows, 16 + is_bob])
        plsc.store_scatter(out_v, [rows * 2], logp)
        plsc.store_scatter(out_v, [rows * 2 + 1], v)
        return _

    lax.fori_loop(0, ROWS_W // 16, group, None)
    pltpu.sync_copy(out_v, out_hbm.at[pl.ds(base * 2, ROWS_W * 2)])


def _make_combine():
    return functools.partial(
        pl.kernel,
        out_type=jax.ShapeDtypeStruct((B * 2,), jnp.float32),
        mesh=plsc.VectorSubcoreMesh(core_axis_name="c", subcore_axis_name="s"),
        scratch_types=[
            pltpu.VMEM((ROWS_W, 128), jnp.float32),
            pltpu.VMEM((ROWS_W,), jnp.int32),
            pltpu.VMEM((ROWS_W * 2,), jnp.float32),
        ],
        compiler_params=pltpu.CompilerParams(needs_layout_passes=False),
    )(_combine_body)


def kernel(x, a, aw1, ab1, aw2, ab2, aw3, ab3, bw1, bb1, bw2, bb2, bw3, bb3,
           acw1, acb1, acw2, acb2, acw3, acb3, bcw1, bcb1, bcw2, bcb2, bcw3,
           bcb3):
    f32 = jnp.float32
    full = lambda s: pl.BlockSpec(s, lambda: (0,) * len(s))
    w1, wt, b1, w2, b2, w3, b3 = pl.pallas_call(
        _assemble_body,
        in_specs=[full(t.shape) for t in (
            aw1, ab1, aw2, ab2, aw3, ab3, bw1, bb1, bw2, bb2, bw3, bb3,
            acw1, acb1, acw2, acb2, acw3, acb3, bcw1, bcb1, bcw2, bcb2,
            bcw3, bcb3)],
        out_specs=[full((MAIN_W, 4 * HID)), full((24, 4 * HID)),
                   full((1, 4 * HID)),
                   full((4 * HID, 4 * HID)), full((1, 4 * HID)),
                   full((4 * HID, ZW)), full((1, ZW))],
        out_shape=[jax.ShapeDtypeStruct((MAIN_W, 4 * HID), jnp.bfloat16),
                   jax.ShapeDtypeStruct((24, 4 * HID), f32),
                   jax.ShapeDtypeStruct((1, 4 * HID), f32),
                   jax.ShapeDtypeStruct((4 * HID, 4 * HID), f32),
                   jax.ShapeDtypeStruct((1, 4 * HID), f32),
                   jax.ShapeDtypeStruct((4 * HID, ZW), f32),
                   jax.ShapeDtypeStruct((1, ZW), f32)],
    )(aw1, ab1, aw2, ab2, aw3, ab3, bw1, bb1, bw2, bb2, bw3, bb3,
      acw1, acb1, acw2, acb2, acw3, acb3, bcw1, bcb1, bcw2, bcb2, bcw3, bcb3)

    xa = lax.slice(x, (0, 0), (B, MAIN_W)).astype(jnp.bfloat16)
    xt = lax.slice(x, (0, MAIN_W), (B, NUM_INPUTS))
    a1 = a.astype(jnp.int32)

    grid = (B // STEP_B,)
    xs = lambda k: pl.BlockSpec((SUB_B, MAIN_W),
                                lambda i, kk=k: (N_STREAMS * i + kk, 0))
    z = pl.pallas_call(
        _dense_body,
        grid=grid,
        in_specs=[
            xs(0), xs(1), xs(2), xs(3),
            pl.BlockSpec((STEP_B, TAIL_W), lambda i: (i, 0)),
            pl.BlockSpec((MAIN_W, 4 * HID), lambda i: (0, 0)),
            pl.BlockSpec((24, 4 * HID), lambda i: (0, 0)),
            pl.BlockSpec((1, 4 * HID), lambda i: (0, 0)),
            pl.BlockSpec((4 * HID, 4 * HID), lambda i: (0, 0)),
            pl.BlockSpec((1, 4 * HID), lambda i: (0, 0)),
            pl.BlockSpec((4 * HID, ZW), lambda i: (0, 0)),
            pl.BlockSpec((1, ZW), lambda i: (0, 0)),
        ],
        out_specs=pl.BlockSpec((STEP_B, 128), lambda i: (i, 0)),
        out_shape=jax.ShapeDtypeStruct((B, 128), f32),
    )(xa, xa, xa, xa, xt, w1, wt, b1, w2, b2, w3, b3)

    return _make_combine()(z, a1).reshape(B, 2)


# speedup vs baseline: 1.2591x; 1.1803x over previous
import functools

import jax
import jax.numpy as jnp
from jax import lax
from jax.experimental import pallas as pl
from jax.experimental.pallas import tpu as pltpu
from jax.experimental.pallas import tpu_sc as plsc

INPUT_DIM = 768
META_DIM = 16
HID = 32
NUM_ACTIONS = 8
NUM_INPUTS = 2 * INPUT_DIM + META_DIM + 1
N_AC = INPUT_DIM + META_DIM
MAIN_W = 1536
TAIL_W = NUM_INPUTS - MAIN_W
ZW = 32
B = 16384
SUB_B = 512
N_STREAMS = 4
STEP_B = SUB_B * N_STREAMS

NW = 32
ROWS_W = B // NW
LN2 = 0.6931471805599453
SQRT2 = 1.4142135623730951


def _assemble_body(aw1, ab1, aw2, ab2, aw3, ab3, bw1, bb1, bw2, bb2, bw3, bb3,
                   acw1, acb1, acw2, acb2, acw3, acb3, bcw1, bcb1, bcw2, bcb2,
                   bcw3, bcb3, w1o, wto, b1o, w2o, b2o, w3o, b3o):
    f32 = jnp.float32
    bf16 = jnp.bfloat16
    w1o[...] = jnp.zeros((MAIN_W, 4 * HID), bf16)
    w1o[0:N_AC, 0:HID] = aw1[...].astype(bf16)
    w1o[0:INPUT_DIM, HID:2 * HID] = bw1[0:INPUT_DIM, :].astype(bf16)
    w1o[N_AC:MAIN_W, HID:2 * HID] = (
        bw1[INPUT_DIM:INPUT_DIM + MAIN_W - N_AC, :].astype(bf16))
    w1o[0:MAIN_W, 2 * HID:3 * HID] = acw1[0:MAIN_W, :].astype(bf16)
    w1o[0:MAIN_W, 3 * HID:4 * HID] = bcw1[0:MAIN_W, :].astype(bf16)
    wto[...] = jnp.zeros((24, 4 * HID), f32)
    wto[0:TAIL_W - 1, HID:2 * HID] = (
        bw1[INPUT_DIM + MAIN_W - N_AC:2 * INPUT_DIM, :].astype(f32))
    wto[0:TAIL_W, 2 * HID:3 * HID] = acw1[MAIN_W:NUM_INPUTS, :]
    wto[0:TAIL_W, 3 * HID:4 * HID] = bcw1[MAIN_W:NUM_INPUTS, :]
    w2o[...] = jnp.zeros((4 * HID, 4 * HID), f32)
    w2o[0:HID, 0:HID] = aw2[...]
    w2o[HID:2 * HID, HID:2 * HID] = bw2[...]
    w2o[2 * HID:3 * HID, 2 * HID:3 * HID] = acw2[...]
    w2o[3 * HID:4 * HID, 3 * HID:4 * HID] = bcw2[...]
    w3o[...] = jnp.zeros((4 * HID, ZW), f32)
    w3o[0:HID, 0:NUM_ACTIONS] = aw3[...]
    w3o[HID:2 * HID, NUM_ACTIONS:2 * NUM_ACTIONS] = bw3[...]
    w3o[2 * HID:3 * HID, 16:17] = acw3[...]
    w3o[3 * HID:4 * HID, 17:18] = bcw3[...]
    b1o[0:1, 0:HID] = ab1[...].reshape(1, HID)
    b1o[0:1, HID:2 * HID] = bb1[...].reshape(1, HID)
    b1o[0:1, 2 * HID:3 * HID] = acb1[...].reshape(1, HID)
    b1o[0:1, 3 * HID:4 * HID] = bcb1[...].reshape(1, HID)
    b2o[0:1, 0:HID] = ab2[...].reshape(1, HID)
    b2o[0:1, HID:2 * HID] = bb2[...].reshape(1, HID)
    b2o[0:1, 2 * HID:3 * HID] = acb2[...].reshape(1, HID)
    b2o[0:1, 3 * HID:4 * HID] = bcb2[...].reshape(1, HID)
    b3o[...] = jnp.zeros((1, ZW), f32)
    b3o[0:1, 0:NUM_ACTIONS] = ab3[...].reshape(1, NUM_ACTIONS)
    b3o[0:1, NUM_ACTIONS:2 * NUM_ACTIONS] = bb3[...].reshape(1, NUM_ACTIONS)
    b3o[0:1, 16:17] = acb3[...].reshape(1, 1)
    b3o[0:1, 17:18] = bcb3[...].reshape(1, 1)


def _dense_body(x0, x1, x2, x3, xt, w1, wt, b1, w2, b2, w3, b3, z_ref):
    for k, x_ref in enumerate((x0, x1, x2, x3)):
        tail = xt[pl.ds(k * SUB_B, SUB_B), :]
        acc = jnp.dot(x_ref[0], w1[0:128, :],
                      preferred_element_type=jnp.float32)
        for j in range(1, MAIN_W // 128):
            acc += jnp.dot(x_ref[j], w1[pl.ds(j * 128, 128), :],
                           preferred_element_type=jnp.float32)
        acc += jnp.dot(tail, wt[0:TAIL_W, :],
                       preferred_element_type=jnp.float32)
        h1 = jnp.tanh(acc + b1[...])
        h2 = jnp.tanh(
            jnp.dot(h1, w2[...], preferred_element_type=jnp.float32) + b2[...])
        z = jnp.dot(h2, w3[...], preferred_element_type=jnp.float32) + b3[...]
        rows = pl.ds(k * SUB_B, SUB_B)
        z_ref[rows, 0:ZW] = z
        z_ref[rows, 18:19] = tail[:, TAIL_W - 1:TAIL_W]


def _ln(s):
    i32 = jnp.int32
    f32 = jnp.float32
    bits = lax.bitcast_convert_type(s, i32)
    k = (bits >> 23) - 127
    man = lax.bitcast_convert_type(
        (bits & jnp.int32(0x007FFFFF)) | jnp.int32(0x3F800000), f32)
    adj = man > SQRT2
    man = jnp.where(adj, man * 0.5, man)
    kf = k.astype(f32) + jnp.where(adj, 1.0, 0.0).astype(f32)
    t = (man - 1.0) / (man + 1.0)
    t2 = t * t
    series = t * (2.0 + t2 * (2.0 / 3.0 + t2 * (2.0 / 5.0 + t2 * (2.0 / 7.0))))
    return kf * LN2 + series


def _combine_body(z_hbm, a_hbm, out_hbm, z_v, a_v, out_v):
    wid = lax.axis_index("s") * 2 + lax.axis_index("c")
    base = wid * ROWS_W
    pltpu.sync_copy(z_hbm.at[pl.ds(base, ROWS_W)], z_v)
    pltpu.sync_copy(a_hbm.at[pl.ds(base, ROWS_W)], a_v)

    lane = lax.iota(jnp.int32, 16)

    def group(g, _):
        rows = g * 16 + lane
        mind = plsc.load_gather(z_v, [rows, jnp.full((16,), 18, jnp.int32)])
        is_bob = (mind > 1.5).astype(jnp.int32)
        col0 = is_bob * NUM_ACTIONS
        l0 = plsc.load_gather(z_v, [rows, col0])
        m = l0
        for j in range(1, NUM_ACTIONS):
            lj = plsc.load_gather(z_v, [rows, col0 + j])
            m = jnp.maximum(m, lj)
        s = jnp.zeros((16,), jnp.float32)
        for j in range(NUM_ACTIONS):
            lj = plsc.load_gather(z_v, [rows, col0 + j])
            s = s + jnp.exp(lj - m)
        lse = m + _ln(s)
        a_vec = a_v[pl.ds(g * 16, 16)]
        sel = plsc.load_gather(z_v, [rows, col0 + a_vec])
        logp = sel - lse
        v = plsc.load_gather(z_v, [rows, 16 + is_bob])
        plsc.store_scatter(out_v, [rows * 2], logp)
        plsc.store_scatter(out_v, [rows * 2 + 1], v)
        return _

    lax.fori_loop(0, ROWS_W // 16, group, None)
    pltpu.sync_copy(out_v, out_hbm.at[pl.ds(base * 2, ROWS_W * 2)])


def _make_combine():
    return functools.partial(
        pl.kernel,
        out_type=jax.ShapeDtypeStruct((B * 2,), jnp.float32),
        mesh=plsc.VectorSubcoreMesh(core_axis_name="c", subcore_axis_name="s"),
        scratch_types=[
            pltpu.VMEM((ROWS_W, 128), jnp.float32),
            pltpu.VMEM((ROWS_W,), jnp.int32),
            pltpu.VMEM((ROWS_W * 2,), jnp.float32),
        ],
        compiler_params=pltpu.CompilerParams(needs_layout_passes=False),
    )(_combine_body)


def kernel(x, a, aw1, ab1, aw2, ab2, aw3, ab3, bw1, bb1, bw2, bb2, bw3, bb3,
           acw1, acb1, acw2, acb2, acw3, acb3, bcw1, bcb1, bcw2, bcb2, bcw3,
           bcb3):
    f32 = jnp.float32
    full = lambda s: pl.BlockSpec(s, lambda: (0,) * len(s))
    w1, wt, b1, w2, b2, w3, b3 = pl.pallas_call(
        _assemble_body,
        in_specs=[full(t.shape) for t in (
            aw1, ab1, aw2, ab2, aw3, ab3, bw1, bb1, bw2, bb2, bw3, bb3,
            acw1, acb1, acw2, acb2, acw3, acb3, bcw1, bcb1, bcw2, bcb2,
            bcw3, bcb3)],
        out_specs=[full((MAIN_W, 4 * HID)), full((24, 4 * HID)),
                   full((1, 4 * HID)),
                   full((4 * HID, 4 * HID)), full((1, 4 * HID)),
                   full((4 * HID, ZW)), full((1, ZW))],
        out_shape=[jax.ShapeDtypeStruct((MAIN_W, 4 * HID), jnp.bfloat16),
                   jax.ShapeDtypeStruct((24, 4 * HID), f32),
                   jax.ShapeDtypeStruct((1, 4 * HID), f32),
                   jax.ShapeDtypeStruct((4 * HID, 4 * HID), f32),
                   jax.ShapeDtypeStruct((1, 4 * HID), f32),
                   jax.ShapeDtypeStruct((4 * HID, ZW), f32),
                   jax.ShapeDtypeStruct((1, ZW), f32)],
    )(aw1, ab1, aw2, ab2, aw3, ab3, bw1, bb1, bw2, bb2, bw3, bb3,
      acw1, acb1, acw2, acb2, acw3, acb3, bcw1, bcb1, bcw2, bcb2, bcw3, bcb3)

    xa = jnp.swapaxes(
        lax.slice(x, (0, 0), (B, MAIN_W)).astype(jnp.bfloat16)
        .reshape(B, MAIN_W // 128, 128), 0, 1)
    xt = lax.slice(x, (0, MAIN_W), (B, NUM_INPUTS))
    a1 = a.astype(jnp.int32)

    grid = (B // STEP_B,)
    xs = lambda k: pl.BlockSpec((MAIN_W // 128, SUB_B, 128),
                                lambda i, kk=k: (0, N_STREAMS * i + kk, 0))
    z = pl.pallas_call(
        _dense_body,
        grid=grid,
        in_specs=[
            xs(0), xs(1), xs(2), xs(3),
            pl.BlockSpec((STEP_B, TAIL_W), lambda i: (i, 0)),
            pl.BlockSpec((MAIN_W, 4 * HID), lambda i: (0, 0)),
            pl.BlockSpec((24, 4 * HID), lambda i: (0, 0)),
            pl.BlockSpec((1, 4 * HID), lambda i: (0, 0)),
            pl.BlockSpec((4 * HID, 4 * HID), lambda i: (0, 0)),
            pl.BlockSpec((1, 4 * HID), lambda i: (0, 0)),
            pl.BlockSpec((4 * HID, ZW), lambda i: (0, 0)),
            pl.BlockSpec((1, ZW), lambda i: (0, 0)),
        ],
        out_specs=pl.BlockSpec((STEP_B, 128), lambda i: (i, 0)),
        out_shape=jax.ShapeDtypeStruct((B, 128), f32),
    )(xa, xa, xa, xa, xt, w1, wt, b1, w2, b2, w3, b3)

    return _make_combine()(z, a1).reshape(B, 2)


# speedup vs baseline: 1.2716x; 1.0099x over previous
import functools

import jax
import jax.numpy as jnp
from jax import lax
from jax.experimental import pallas as pl
from jax.experimental.pallas import tpu as pltpu
from jax.experimental.pallas import tpu_sc as plsc

INPUT_DIM = 768
META_DIM = 16
HID = 32
NUM_ACTIONS = 8
NUM_INPUTS = 2 * INPUT_DIM + META_DIM + 1
N_AC = INPUT_DIM + META_DIM
MAIN_W = 1536
TAIL_W = NUM_INPUTS - MAIN_W
ZW = 32
B = 16384
SUB_B = 1024
N_STREAMS = 4
STEP_B = SUB_B * N_STREAMS

NW = 32
ROWS_W = B // NW
LN2 = 0.6931471805599453
SQRT2 = 1.4142135623730951


def _assemble_body(aw1, ab1, aw2, ab2, aw3, ab3, bw1, bb1, bw2, bb2, bw3, bb3,
                   acw1, acb1, acw2, acb2, acw3, acb3, bcw1, bcb1, bcw2, bcb2,
                   bcw3, bcb3, w1o, wto, b1o, w2o, b2o, w3o, b3o):
    f32 = jnp.float32
    bf16 = jnp.bfloat16
    w1o[...] = jnp.zeros((MAIN_W, 4 * HID), bf16)
    w1o[0:N_AC, 0:HID] = aw1[...].astype(bf16)
    w1o[0:INPUT_DIM, HID:2 * HID] = bw1[0:INPUT_DIM, :].astype(bf16)
    w1o[N_AC:MAIN_W, HID:2 * HID] = (
        bw1[INPUT_DIM:INPUT_DIM + MAIN_W - N_AC, :].astype(bf16))
    w1o[0:MAIN_W, 2 * HID:3 * HID] = acw1[0:MAIN_W, :].astype(bf16)
    w1o[0:MAIN_W, 3 * HID:4 * HID] = bcw1[0:MAIN_W, :].astype(bf16)
    wto[...] = jnp.zeros((24, 4 * HID), f32)
    wto[0:TAIL_W - 1, HID:2 * HID] = (
        bw1[INPUT_DIM + MAIN_W - N_AC:2 * INPUT_DIM, :].astype(f32))
    wto[0:TAIL_W, 2 * HID:3 * HID] = acw1[MAIN_W:NUM_INPUTS, :]
    wto[0:TAIL_W, 3 * HID:4 * HID] = bcw1[MAIN_W:NUM_INPUTS, :]
    w2o[...] = jnp.zeros((4 * HID, 4 * HID), f32)
    w2o[0:HID, 0:HID] = aw2[...]
    w2o[HID:2 * HID, HID:2 * HID] = bw2[...]
    w2o[2 * HID:3 * HID, 2 * HID:3 * HID] = acw2[...]
    w2o[3 * HID:4 * HID, 3 * HID:4 * HID] = bcw2[...]
    w3o[...] = jnp.zeros((4 * HID, ZW), f32)
    w3o[0:HID, 0:NUM_ACTIONS] = aw3[...]
    w3o[HID:2 * HID, NUM_ACTIONS:2 * NUM_ACTIONS] = bw3[...]
    w3o[2 * HID:3 * HID, 16:17] = acw3[...]
    w3o[3 * HID:4 * HID, 17:18] = bcw3[...]
    b1o[0:1, 0:HID] = ab1[...].reshape(1, HID)
    b1o[0:1, HID:2 * HID] = bb1[...].reshape(1, HID)
    b1o[0:1, 2 * HID:3 * HID] = acb1[...].reshape(1, HID)
    b1o[0:1, 3 * HID:4 * HID] = bcb1[...].reshape(1, HID)
    b2o[0:1, 0:HID] = ab2[...].reshape(1, HID)
    b2o[0:1, HID:2 * HID] = bb2[...].reshape(1, HID)
    b2o[0:1, 2 * HID:3 * HID] = acb2[...].reshape(1, HID)
    b2o[0:1, 3 * HID:4 * HID] = bcb2[...].reshape(1, HID)
    b3o[...] = jnp.zeros((1, ZW), f32)
    b3o[0:1, 0:NUM_ACTIONS] = ab3[...].reshape(1, NUM_ACTIONS)
    b3o[0:1, NUM_ACTIONS:2 * NUM_ACTIONS] = bb3[...].reshape(1, NUM_ACTIONS)
    b3o[0:1, 16:17] = acb3[...].reshape(1, 1)
    b3o[0:1, 17:18] = bcb3[...].reshape(1, 1)


def _dense_body(x0, x1, x2, x3, xt, w1, wt, b1, w2, b2, w3, b3, z_ref):
    for k, x_ref in enumerate((x0, x1, x2, x3)):
        tail = xt[pl.ds(k * SUB_B, SUB_B), :]
        acc = jnp.dot(x_ref[0], w1[0:128, :],
                      preferred_element_type=jnp.float32)
        for j in range(1, MAIN_W // 128):
            acc += jnp.dot(x_ref[j], w1[pl.ds(j * 128, 128), :],
                           preferred_element_type=jnp.float32)
        acc += jnp.dot(tail, wt[0:TAIL_W, :],
                       preferred_element_type=jnp.float32)
        h1 = jnp.tanh(acc + b1[...])
        h2 = jnp.tanh(
            jnp.dot(h1, w2[...], preferred_element_type=jnp.float32) + b2[...])
        z = jnp.dot(h2, w3[...], preferred_element_type=jnp.float32) + b3[...]
        rows = pl.ds(k * SUB_B, SUB_B)
        z_ref[rows, 0:ZW] = z
        z_ref[rows, 18:19] = tail[:, TAIL_W - 1:TAIL_W]


def _ln(s):
    i32 = jnp.int32
    f32 = jnp.float32
    bits = lax.bitcast_convert_type(s, i32)
    k = (bits >> 23) - 127
    man = lax.bitcast_convert_type(
        (bits & jnp.int32(0x007FFFFF)) | jnp.int32(0x3F800000), f32)
    adj = man > SQRT2
    man = jnp.where(adj, man * 0.5, man)
    kf = k.astype(f32) + jnp.where(adj, 1.0, 0.0).astype(f32)
    t = (man - 1.0) / (man + 1.0)
    t2 = t * t
    series = t * (2.0 + t2 * (2.0 / 3.0 + t2 * (2.0 / 5.0 + t2 * (2.0 / 7.0))))
    return kf * LN2 + series


def _combine_body(z_hbm, a_hbm, out_hbm, z_v, a_v, out_v):
    wid = lax.axis_index("s") * 2 + lax.axis_index("c")
    base = wid * ROWS_W
    pltpu.sync_copy(z_hbm.at[pl.ds(base, ROWS_W)], z_v)
    pltpu.sync_copy(a_hbm.at[pl.ds(base, ROWS_W)], a_v)

    lane = lax.iota(jnp.int32, 16)

    def group(g, _):
        rows = g * 16 + lane
        mind = plsc.load_gather(z_v, [rows, jnp.full((16,), 18, jnp.int32)])
        is_bob = (mind > 1.5).astype(jnp.int32)
        col0 = is_bob * NUM_ACTIONS
        l0 = plsc.load_gather(z_v, [rows, col0])
        m = l0
        for j in range(1, NUM_ACTIONS):
            lj = plsc.load_gather(z_v, [rows, col0 + j])
            m = jnp.maximum(m, lj)
        s = jnp.zeros((16,), jnp.float32)
        for j in range(NUM_ACTIONS):
            lj = plsc.load_gather(z_v, [rows, col0 + j])
            s = s + jnp.exp(lj - m)
        lse = m + _ln(s)
        a_vec = a_v[pl.ds(g * 16, 16)]
        sel = plsc.load_gather(z_v, [rows, col0 + a_vec])
        logp = sel - lse
        v = plsc.load_gather(z_v, [rows, 16 + is_bob])
        plsc.store_scatter(out_v, [rows * 2], logp)
        plsc.store_scatter(out_v, [rows * 2 + 1], v)
        return _

    lax.fori_loop(0, ROWS_W // 16, group, None)
    pltpu.sync_copy(out_v, out_hbm.at[pl.ds(base * 2, ROWS_W * 2)])


def _make_combine():
    return functools.partial(
        pl.kernel,
        out_type=jax.ShapeDtypeStruct((B * 2,), jnp.float32),
        mesh=plsc.VectorSubcoreMesh(core_axis_name="c", subcore_axis_name="s"),
        scratch_types=[
            pltpu.VMEM((ROWS_W, 128), jnp.float32),
            pltpu.VMEM((ROWS_W,), jnp.int32),
            pltpu.VMEM((ROWS_W * 2,), jnp.float32),
        ],
        compiler_params=pltpu.CompilerParams(needs_layout_passes=False),
    )(_combine_body)


def kernel(x, a, aw1, ab1, aw2, ab2, aw3, ab3, bw1, bb1, bw2, bb2, bw3, bb3,
           acw1, acb1, acw2, acb2, acw3, acb3, bcw1, bcb1, bcw2, bcb2, bcw3,
           bcb3):
    f32 = jnp.float32
    full = lambda s: pl.BlockSpec(s, lambda: (0,) * len(s))
    w1, wt, b1, w2, b2, w3, b3 = pl.pallas_call(
        _assemble_body,
        in_specs=[full(t.shape) for t in (
            aw1, ab1, aw2, ab2, aw3, ab3, bw1, bb1, bw2, bb2, bw3, bb3,
            acw1, acb1, acw2, acb2, acw3, acb3, bcw1, bcb1, bcw2, bcb2,
            bcw3, bcb3)],
        out_specs=[full((MAIN_W, 4 * HID)), full((24, 4 * HID)),
                   full((1, 4 * HID)),
                   full((4 * HID, 4 * HID)), full((1, 4 * HID)),
                   full((4 * HID, ZW)), full((1, ZW))],
        out_shape=[jax.ShapeDtypeStruct((MAIN_W, 4 * HID), jnp.bfloat16),
                   jax.ShapeDtypeStruct((24, 4 * HID), f32),
                   jax.ShapeDtypeStruct((1, 4 * HID), f32),
                   jax.ShapeDtypeStruct((4 * HID, 4 * HID), f32),
                   jax.ShapeDtypeStruct((1, 4 * HID), f32),
                   jax.ShapeDtypeStruct((4 * HID, ZW), f32),
                   jax.ShapeDtypeStruct((1, ZW), f32)],
    )(aw1, ab1, aw2, ab2, aw3, ab3, bw1, bb1, bw2, bb2, bw3, bb3,
      acw1, acb1, acw2, acb2, acw3, acb3, bcw1, bcb1, bcw2, bcb2, bcw3, bcb3)

    xa = jnp.swapaxes(
        lax.slice(x, (0, 0), (B, MAIN_W)).astype(jnp.bfloat16)
        .reshape(B, MAIN_W // 128, 128), 0, 1)
    xt = lax.slice(x, (0, MAIN_W), (B, NUM_INPUTS))
    a1 = a.astype(jnp.int32)

    grid = (B // STEP_B,)
    xs = lambda k: pl.BlockSpec((MAIN_W // 128, SUB_B, 128),
                                lambda i, kk=k: (0, N_STREAMS * i + kk, 0))
    z = pl.pallas_call(
        _dense_body,
        grid=grid,
        in_specs=[
            xs(0), xs(1), xs(2), xs(3),
            pl.BlockSpec((STEP_B, TAIL_W), lambda i: (i, 0)),
            pl.BlockSpec((MAIN_W, 4 * HID), lambda i: (0, 0)),
            pl.BlockSpec((24, 4 * HID), lambda i: (0, 0)),
            pl.BlockSpec((1, 4 * HID), lambda i: (0, 0)),
            pl.BlockSpec((4 * HID, 4 * HID), lambda i: (0, 0)),
            pl.BlockSpec((1, 4 * HID), lambda i: (0, 0)),
            pl.BlockSpec((4 * HID, ZW), lambda i: (0, 0)),
            pl.BlockSpec((1, ZW), lambda i: (0, 0)),
        ],
        out_specs=pl.BlockSpec((STEP_B, 128), lambda i: (i, 0)),
        out_shape=jax.ShapeDtypeStruct((B, 128), f32),
    )(xa, xa, xa, xa, xt, w1, wt, b1, w2, b2, w3, b3)

    return _make_combine()(z, a1).reshape(B, 2)
